# Initial kernel scaffold; baseline (speedup 1.0000x reference)
import jax
import jax.numpy as jnp
from jax.experimental import pallas as pl


def kernel(x, W):
    raise NotImplementedError("write your pallas kernel here")


# speedup vs baseline: 5.2142x; 5.2142x over previous
import functools

import jax
import jax.numpy as jnp
from jax import lax
from jax.experimental import pallas as pl
from jax.experimental.pallas import tpu as pltpu
from jax.experimental.pallas import tpu_sc as plsc

_DIM = 64
_CHUNK = 128

_info = plsc.get_sparse_core_info()
_NC = _info.num_cores
_NS = _info.num_subcores
_NW = _NC * _NS


@functools.partial(jax.jit, static_argnames=("n_rows",))
def _gather_rows(xf, W, n_rows):
    n_per_w = n_rows // _NW
    n_chunks = n_per_w // _CHUNK
    mesh = plsc.VectorSubcoreMesh(core_axis_name="c", subcore_axis_name="s")

    @functools.partial(
        pl.kernel,
        mesh=mesh,
        compiler_params=pltpu.CompilerParams(use_tc_tiling_on_sc=False),
        out_type=jax.ShapeDtypeStruct((n_rows, _DIM), jnp.float32),
        scratch_types=[
            pltpu.VMEM((n_chunks, _CHUNK), jnp.int32),
            pltpu.VMEM((_CHUNK, _DIM), jnp.float32),
            pltpu.SemaphoreType.DMA,
            pltpu.SemaphoreType.DMA,
        ],
    )
    def k(idx_hbm, table_hbm, out_hbm, idx_v, rows_v, sem_g, sem_o):
        wid = lax.axis_index("s") * _NC + lax.axis_index("c")
        base = wid * n_per_w
        pltpu.sync_copy(idx_hbm.at[wid], idx_v)

        def body(j, carry):
            pltpu.async_copy(table_hbm.at[idx_v.at[j]], rows_v, sem_g).wait()
            pltpu.async_copy(
                rows_v, out_hbm.at[pl.ds(base + j * _CHUNK, _CHUNK)], sem_o
            ).wait()
            return carry

        lax.fori_loop(0, n_chunks, body, 0)

    return k(xf, W)


def kernel(x, W):
    n_rows = x.shape[0] * x.shape[1]
    xf = x.astype(jnp.int32).reshape(_NW, (n_rows // _NW) // _CHUNK, _CHUNK)
    out = _gather_rows(xf, W, n_rows)
    return out.reshape(x.shape[0], x.shape[1], _DIM)


# speedup vs baseline: 6.0942x; 1.1688x over previous
import functools

import jax
import jax.numpy as jnp
from jax import lax
from jax.experimental import pallas as pl
from jax.experimental.pallas import tpu as pltpu
from jax.experimental.pallas import tpu_sc as plsc

_DIM = 64
_CHUNK = 128
_NBUF = 8
_LEAD = 4

_info = plsc.get_sparse_core_info()
_NC = _info.num_cores
_NS = _info.num_subcores
_NW = _NC * _NS


@functools.partial(jax.jit, static_argnames=("n_rows",))
def _gather_rows(xf, W, n_rows):
    n_per_w = n_rows // _NW
    n_chunks = n_per_w // _CHUNK
    assert n_chunks % _NBUF == 0 and n_chunks >= 2 * _NBUF
    mesh = plsc.VectorSubcoreMesh(core_axis_name="c", subcore_axis_name="s")

    @functools.partial(
        pl.kernel,
        mesh=mesh,
        compiler_params=pltpu.CompilerParams(use_tc_tiling_on_sc=False),
        out_type=jax.ShapeDtypeStruct((n_rows, _DIM), jnp.float32),
        scratch_types=(
            [pltpu.VMEM((n_chunks, _CHUNK), jnp.int32)]
            + [pltpu.VMEM((_CHUNK, _DIM), jnp.float32)] * _NBUF
            + [pltpu.SemaphoreType.DMA] * (2 * _NBUF)
        ),
    )
    def k(idx_hbm, table_hbm, out_hbm, idx_v, *bufs):
        rows = bufs[:_NBUF]
        sem_g = bufs[_NBUF : 2 * _NBUF]
        sem_o = bufs[2 * _NBUF :]
        wid = lax.axis_index("s") * _NC + lax.axis_index("c")
        base = wid * n_per_w
        pltpu.sync_copy(idx_hbm.at[wid], idx_v)

        def fire_gather(f, s):
            pltpu.async_copy(table_hbm.at[idx_v.at[f]], rows[s], sem_g[s])

        def wait_gather(j, s):
            pltpu.make_async_copy(table_hbm.at[idx_v.at[j]], rows[s], sem_g[s]).wait()

        def fire_out(j, s):
            pltpu.async_copy(
                rows[s], out_hbm.at[pl.ds(base + j * _CHUNK, _CHUNK)], sem_o[s]
            )

        def wait_out(j, s):
            pltpu.make_async_copy(
                rows[s], out_hbm.at[pl.ds(base + j * _CHUNK, _CHUNK)], sem_o[s]
            ).wait()

        for f in range(_LEAD):
            fire_gather(f, f)

        def body(m, carry):
            for b in range(_NBUF):
                j = m * _NBUF + b
                f = j + _LEAD
                fs = (b + _LEAD) % _NBUF

                @pl.when(f < n_chunks)
                def _():
                    @pl.when(f >= _NBUF)
                    def _():
                        wait_out(f - _NBUF, fs)

                    fire_gather(f, fs)

                wait_gather(j, b)
                fire_out(j, b)
            return carry

        lax.fori_loop(0, n_chunks // _NBUF, body, 0)
        for b in range(_NBUF):
            wait_out(n_chunks - _NBUF + b, b)

    return k(xf, W)


def kernel(x, W):
    n_rows = x.shape[0] * x.shape[1]
    xf = x.astype(jnp.int32).reshape(_NW, (n_rows // _NW) // _CHUNK, _CHUNK)
    out = _gather_rows(xf, W, n_rows)
    return out.reshape(x.shape[0], x.shape[1], _DIM)
